# Initial kernel scaffold; baseline (speedup 1.0000x reference)
import jax
import jax.numpy as jnp
from jax.experimental import pallas as pl


def kernel(xyzs, dirs, ts, cu_seqlens, W1, W2):
    raise NotImplementedError("write your pallas kernel here")


# speedup vs baseline: 180.5962x; 180.5962x over previous
import functools

import jax
import jax.numpy as jnp
from jax import lax
from jax.experimental import pallas as pl
from jax.experimental.pallas import tpu as pltpu
from jax.experimental.pallas import tpu_sc as plsc

T = 1048576
N = 65536

BLK = 2048
RR = BLK // 128
NB = T // BLK
NCHUNK = T // 128
NGRP = NCHUNK // 128

NTILE = 32
RPT = N // NTILE
CH = 2048
BIG = 3.0e38

HT = T // 2
SLC = HT // 16
EPS0 = N // 16


def _sc_counts(cu_pad):
    mesh = plsc.VectorSubcoreMesh(core_axis_name="c", subcore_axis_name="s")

    @functools.partial(
        pl.kernel,
        out_type=jax.ShapeDtypeStruct((T,), jnp.int32),
        mesh=mesh,
        scratch_types=[
            pltpu.VMEM((EPS0,), jnp.int32),
            pltpu.VMEM((EPS0,), jnp.int32),
            pltpu.VMEM((EPS0,), jnp.int32),
            pltpu.VMEM((SLC,), jnp.int32),
            pltpu.VMEM_SHARED((HT + 256,), jnp.int32),
        ],
        compiler_params=pltpu.CompilerParams(needs_layout_passes=False),
    )
    def k(cu_hbm, out_hbm, idx_v, idx2_v, ones_v, zbuf_v, cnt_sh):
        cid = lax.axis_index("c")
        sid = lax.axis_index("s")
        z16 = jnp.zeros((16,), jnp.int32)
        o16 = jnp.ones((16,), jnp.int32)

        def fill(i, _):
            zbuf_v[pl.ds(i * 16, 16)] = z16
            return 0

        lax.fori_loop(0, SLC // 16, fill, 0)

        def fill2(i, _):
            ones_v[pl.ds(i * 16, 16)] = o16
            return 0

        lax.fori_loop(0, EPS0 // 16, fill2, 0)

        pltpu.sync_copy(zbuf_v, cnt_sh.at[pl.ds(sid * SLC, SLC)])

        @pl.when(sid == 0)
        def _():
            pltpu.sync_copy(zbuf_v.at[pl.ds(0, 256)],
                            cnt_sh.at[pl.ds(HT, 256)])

        plsc.subcore_barrier()

        pltpu.sync_copy(cu_hbm.at[pl.ds(sid * EPS0, EPS0)], idx_v)
        lo = cid * HT

        def redirect(j, _):
            v = idx_v[pl.ds(j * 16, 16)]
            rel = v - lo
            msk = (rel >= 0) & (rel < HT)
            idx2_v[pl.ds(j * 16, 16)] = jnp.where(msk, rel, HT)
            return 0

        lax.fori_loop(0, EPS0 // 16, redirect, 0)
        pltpu.sync_copy(ones_v, cnt_sh.at[idx2_v], add=True)
        plsc.subcore_barrier()

        pltpu.sync_copy(
            cnt_sh.at[pl.ds(sid * SLC, SLC)],
            out_hbm.at[pl.ds(cid * HT + sid * SLC, SLC)],
        )

    return k(cu_pad)


def _ptwise_kernel(xyz_r, dir_r, ts_r, w1_r, w2_r, g_r, p_r):
    x = xyz_r[0, 0]
    y = xyz_r[1, 0]
    z = xyz_r[2, 0]
    dx = dir_r[0, 0]
    dy = dir_r[1, 0]
    dz = dir_r[2, 0]
    t0 = ts_r[0, 0]
    dt = ts_r[1, 0]

    d2 = (x * x + z * z) + y * y
    sig = 10.0 * jnp.exp(d2 * (-2.0))
    e2 = jnp.exp(-sig * dt)
    alpha = 1.0 - e2
    g_r[0] = jnp.log(jnp.clip(1.0 - alpha, 1e-10, 1.0))

    p_r[0, 0] = alpha
    p_r[1, 0] = alpha * t0
    for cch in range(3):
        zc = (x * w1_r[0, cch] + y * w1_r[1, cch] + z * w1_r[2, cch]
              + dx * w2_r[0, cch] + dy * w2_r[1, cch] + dz * w2_r[2, cch])
        rgb = 1.0 / (1.0 + jnp.exp(-zc))
        p_r[2 + cch, 0] = alpha * (rgb - 1.0)


def _tc_pointwise(xyz_t, dir_t, ts_t, w1, w2):
    bs3 = pl.BlockSpec((3, 1, RR, 128), lambda i: (0, i, 0, 0))
    bs2 = pl.BlockSpec((2, 1, RR, 128), lambda i: (0, i, 0, 0))
    bs_sm = pl.BlockSpec(memory_space=pltpu.SMEM)
    return pl.pallas_call(
        _ptwise_kernel,
        grid=(NB,),
        in_specs=[bs3, bs3, bs2, bs_sm, bs_sm],
        out_specs=(
            pl.BlockSpec((1, RR, 128), lambda i: (i, 0, 0)),
            pl.BlockSpec((5, 1, RR, 128), lambda i: (0, i, 0, 0)),
        ),
        out_shape=(
            jax.ShapeDtypeStruct((NB, RR, 128), jnp.float32),
            jax.ShapeDtypeStruct((5, NB, RR, 128), jnp.float32),
        ),
    )(xyz_t, dir_t, ts_t, w1, w2)


def _scan1_kernel(x_r, o_r, acc):
    j = pl.program_id(0)

    @pl.when(j == 0)
    def _():
        acc[...] = jnp.zeros_like(acc)

    acc[...] = acc[...] + x_r[0]
    o_r[0] = acc[...]


def _tc_scan1(x2):
    x3 = x2.reshape(128, NCHUNK // 128, 128)
    return pl.pallas_call(
        _scan1_kernel,
        grid=(128,),
        in_specs=[pl.BlockSpec((1, NCHUNK // 128, 128), lambda j: (j, 0, 0))],
        out_specs=pl.BlockSpec((1, NCHUNK // 128, 128), lambda j: (j, 0, 0)),
        out_shape=jax.ShapeDtypeStruct((128, NCHUNK // 128, 128), jnp.float32),
        scratch_shapes=[pltpu.VMEM((NCHUNK // 128, 128), jnp.float32)],
        compiler_params=pltpu.CompilerParams(
            dimension_semantics=("arbitrary",),
        ),
    )(x3).reshape(128, NCHUNK)


def _scan23_kernel(rs_r, p1_r, acc):
    def step(j, _):
        acc[0, :] = acc[0, :] + rs_r[j, :]
        p1_r[j, :] = acc[0, :]
        return 0

    acc[0, :] = jnp.zeros((NGRP,), jnp.float32)
    lax.fori_loop(0, 128, step, 0)

    lane = lax.broadcasted_iota(jnp.int32, (1, NGRP), 1)
    vals = acc[0:1, :]

    def shr(a):
        return jnp.concatenate(
            [jnp.zeros((1, 1), jnp.float32), a[:, : NGRP - 1]], axis=1)

    for g in range(1, NGRP):
        vals = vals + jnp.where(lane == g, shr(vals), 0.0)
    e3 = shr(vals)
    p1_r[...] = p1_r[...] + e3


def _tc_scan23(rs2):
    return pl.pallas_call(
        _scan23_kernel,
        in_specs=[pl.BlockSpec((128, NGRP), lambda: (0, 0))],
        out_specs=pl.BlockSpec((128, NGRP), lambda: (0, 0)),
        out_shape=jax.ShapeDtypeStruct((128, NGRP), jnp.float32),
        scratch_shapes=[pltpu.VMEM((1, NGRP), jnp.float32)],
    )(rs2)


def _combine_kernel(i1_r, e_r, g_r, cnt_r, ex_r, seg_r, scr):
    i = pl.program_id(0)

    @pl.when(i == 0)
    def _():
        scr[0] = 0.0

    cs = i1_r[0] + e_r[0, 0][:, None]
    ex_r[0] = cs - g_r[0]

    c = cnt_r[0]
    cf = c.astype(jnp.float32)
    ii = lax.broadcasted_iota(jnp.int32, (128, 128), 0)
    jj = lax.broadcasted_iota(jnp.int32, (128, 128), 1)
    ut = (ii <= jj).astype(jnp.float32)
    ri = lax.broadcasted_iota(jnp.int32, (RR, RR), 0)
    rj = lax.broadcasted_iota(jnp.int32, (RR, RR), 1)
    lt = (rj < ri).astype(jnp.float32)
    cl = jnp.dot(cf, ut, preferred_element_type=jnp.float32)
    coffs = jnp.dot(lt, cl[:, 127:128], preferred_element_type=jnp.float32)
    cinc = cl + coffs
    seg_r[0] = (cinc + scr[0] - 1.0).astype(jnp.int32)
    scr[0] = scr[0] + jnp.sum(cf)


def _tc_combine(incl1, e3d, log1m, cnt):
    bs = pl.BlockSpec((1, RR, 128), lambda i: (i, 0, 0))
    return pl.pallas_call(
        _combine_kernel,
        grid=(NB,),
        in_specs=[bs, pl.BlockSpec((1, 1, RR), lambda i: (i, 0, 0)), bs, bs],
        out_specs=(bs, bs),
        out_shape=(
            jax.ShapeDtypeStruct((NB, RR, 128), jnp.float32),
            jax.ShapeDtypeStruct((NB, RR, 128), jnp.int32),
        ),
        scratch_shapes=[pltpu.SMEM((1,), jnp.float32)],
        compiler_params=pltpu.CompilerParams(
            dimension_semantics=("arbitrary",),
        ),
    )(incl1, e3d, log1m, cnt)


def _sc_reduce(excl, chans, seg, cu_pad):
    mesh = plsc.VectorSubcoreMesh(core_axis_name="c", subcore_axis_name="s")
    out_type = (
        jax.ShapeDtypeStruct((3 * N,), jnp.float32),
        jax.ShapeDtypeStruct((N,), jnp.float32),
        jax.ShapeDtypeStruct((N,), jnp.float32),
    )

    @functools.partial(
        pl.kernel,
        out_type=out_type,
        mesh=mesh,
        scratch_types=[
            pltpu.VMEM((16,), jnp.int32),
            pltpu.VMEM((16,), jnp.int32),
            pltpu.VMEM((RPT,), jnp.int32),
            pltpu.VMEM((RPT,), jnp.float32),
            pltpu.VMEM((CH,), jnp.int32),
            pltpu.VMEM((CH,), jnp.float32),
            [pltpu.VMEM((CH,), jnp.float32) for _ in range(5)],
            [pltpu.VMEM((RPT,), jnp.float32) for _ in range(5)],
            pltpu.SemaphoreType.DMA,
        ],
        compiler_params=pltpu.CompilerParams(needs_layout_passes=False),
    )
    def k(ex_hbm, ch_hbm, seg_hbm, cu_hbm, img_hbm, dep_hbm, ws_hbm,
          lo_v, hi_v, st_v, base_v, seg_v, ex_v, chbufs, accs, sem):
        cid = lax.axis_index("c")
        sid = lax.axis_index("s")
        w = cid * 16 + sid
        base_ray = w * RPT

        pltpu.sync_copy(cu_hbm.at[pl.ds(w * RPT, 16)], lo_v)
        pltpu.sync_copy(cu_hbm.at[pl.ds((w + 1) * RPT, 16)], hi_v)
        lo = lo_v[...][0]
        hi = hi_v[...][0]
        k0 = lo // CH
        k1 = (hi + CH - 1) // CH

        pltpu.sync_copy(cu_hbm.at[pl.ds(w * RPT, RPT)], st_v)
        pltpu.async_copy(ex_hbm.at[st_v], base_v, sem).wait()

        z16 = jnp.zeros((16,), jnp.float32)

        def zero(i, _):
            for a in accs:
                a[pl.ds(i * 16, 16)] = z16
            return 0

        lax.fori_loop(0, RPT // 16, zero, 0)

        def chunk(kc, _):
            pltpu.sync_copy(seg_hbm.at[pl.ds(kc * CH, CH)], seg_v)
            pltpu.sync_copy(ex_hbm.at[pl.ds(kc * CH, CH)], ex_v)
            for ci in range(5):
                pltpu.sync_copy(ch_hbm.at[pl.ds(ci * T + kc * CH, CH)],
                                chbufs[ci])

            def inner(j, _):
                s = seg_v[pl.ds(j * 16, 16)]
                rel = s - base_ray
                msk = (rel >= 0) & (rel < RPT)
                relc = jnp.where(msk, rel, 0)
                b = plsc.load_gather(base_v, [relc])
                trans = jnp.exp(ex_v[pl.ds(j * 16, 16)] - b)
                for ci in range(5):
                    v = chbufs[ci][pl.ds(j * 16, 16)] * trans
                    plsc.addupdate_scatter(accs[ci], [relc], v, mask=msk)
                return 0

            lax.fori_loop(0, CH // 16, inner, 0)
            return 0

        lax.fori_loop(k0, k1, chunk, 0)

        one16 = jnp.ones((16,), jnp.float32)

        def addone(i, _):
            for a in accs[2:]:
                a[pl.ds(i * 16, 16)] = a[pl.ds(i * 16, 16)] + one16
            return 0

        lax.fori_loop(0, RPT // 16, addone, 0)

        pltpu.sync_copy(accs[0], ws_hbm.at[pl.ds(base_ray, RPT)])
        pltpu.sync_copy(accs[1], dep_hbm.at[pl.ds(base_ray, RPT)])
        for ci in range(3):
            pltpu.sync_copy(accs[2 + ci],
                            img_hbm.at[pl.ds(ci * N + base_ray, RPT)])

    return k(excl, chans, seg, cu_pad)


def kernel(xyzs, dirs, ts, cu_seqlens, W1, W2):
    xyz_t = xyzs.T.reshape(3, NB, RR, 128)
    dir_t = dirs.T.reshape(3, NB, RR, 128)
    ts_t = ts.T.reshape(2, NB, RR, 128)
    cu_pad = jnp.concatenate(
        [cu_seqlens, jnp.full((15,), T, jnp.int32)])

    cnt = _sc_counts(cu_pad).reshape(NB, RR, 128)
    log1m, chans = _tc_pointwise(xyz_t, dir_t, ts_t, W1, W2)

    x2 = log1m.reshape(NCHUNK, 128).T
    cs1t = _tc_scan1(x2)
    incl1 = cs1t.T.reshape(NB, RR, 128)
    rs2 = cs1t[127].reshape(NGRP, 128).T
    p1t = _tc_scan23(rs2)
    p1 = p1t.T.reshape(NCHUNK)
    e = jnp.concatenate([jnp.zeros((1,), jnp.float32), p1[:-1]])
    e3d = e.reshape(NB, 1, RR)

    excl, seg = _tc_combine(incl1, e3d, log1m, cnt)

    img_t, depth, ws = _sc_reduce(
        excl.reshape(T), chans.reshape(5 * T), seg.reshape(T), cu_pad)
    return img_t.reshape(3, N).T, depth, ws


# speedup vs baseline: 190.3835x; 1.0542x over previous
import functools

import jax
import jax.numpy as jnp
from jax import lax
from jax.experimental import pallas as pl
from jax.experimental.pallas import tpu as pltpu
from jax.experimental.pallas import tpu_sc as plsc

T = 1048576
N = 65536

BLK = 2048
RR = BLK // 128
NB = T // BLK
NCHUNK = T // 128
NGRP = NCHUNK // 128

NTILE = 32
RPT = N // NTILE
CH = 4096
BIG = 3.0e38

HT = T // 2
SLC = HT // 16
EPS0 = N // 16


def _sc_counts(cu_pad):
    mesh = plsc.VectorSubcoreMesh(core_axis_name="c", subcore_axis_name="s")

    @functools.partial(
        pl.kernel,
        out_type=jax.ShapeDtypeStruct((T,), jnp.int32),
        mesh=mesh,
        scratch_types=[
            pltpu.VMEM((EPS0,), jnp.int32),
            pltpu.VMEM((EPS0,), jnp.int32),
            pltpu.VMEM((EPS0,), jnp.int32),
            pltpu.VMEM((SLC,), jnp.int32),
            pltpu.VMEM_SHARED((HT + 256,), jnp.int32),
        ],
        compiler_params=pltpu.CompilerParams(needs_layout_passes=False),
    )
    def k(cu_hbm, out_hbm, idx_v, idx2_v, ones_v, zbuf_v, cnt_sh):
        cid = lax.axis_index("c")
        sid = lax.axis_index("s")
        z16 = jnp.zeros((16,), jnp.int32)
        o16 = jnp.ones((16,), jnp.int32)

        def fill(i, _):
            zbuf_v[pl.ds(i * 16, 16)] = z16
            return 0

        lax.fori_loop(0, SLC // 16, fill, 0)

        def fill2(i, _):
            ones_v[pl.ds(i * 16, 16)] = o16
            return 0

        lax.fori_loop(0, EPS0 // 16, fill2, 0)

        pltpu.sync_copy(zbuf_v, cnt_sh.at[pl.ds(sid * SLC, SLC)])

        @pl.when(sid == 0)
        def _():
            pltpu.sync_copy(zbuf_v.at[pl.ds(0, 256)],
                            cnt_sh.at[pl.ds(HT, 256)])

        plsc.subcore_barrier()

        pltpu.sync_copy(cu_hbm.at[pl.ds(sid * EPS0, EPS0)], idx_v)
        lo = cid * HT

        def redirect(j, _):
            v = idx_v[pl.ds(j * 16, 16)]
            rel = v - lo
            msk = (rel >= 0) & (rel < HT)
            idx2_v[pl.ds(j * 16, 16)] = jnp.where(msk, rel, HT)
            return 0

        lax.fori_loop(0, EPS0 // 16, redirect, 0)
        pltpu.sync_copy(ones_v, cnt_sh.at[idx2_v], add=True)
        plsc.subcore_barrier()

        pltpu.sync_copy(
            cnt_sh.at[pl.ds(sid * SLC, SLC)],
            out_hbm.at[pl.ds(cid * HT + sid * SLC, SLC)],
        )

    return k(cu_pad)


def _ptwise_kernel(xyz_r, dir_r, ts_r, w1_r, w2_r, g_r, p_r):
    x = xyz_r[0, 0]
    y = xyz_r[1, 0]
    z = xyz_r[2, 0]
    dx = dir_r[0, 0]
    dy = dir_r[1, 0]
    dz = dir_r[2, 0]
    t0 = ts_r[0, 0]
    dt = ts_r[1, 0]

    d2 = (x * x + z * z) + y * y
    sig = 10.0 * jnp.exp(d2 * (-2.0))
    e2 = jnp.exp(-sig * dt)
    alpha = 1.0 - e2
    g_r[0] = jnp.log(jnp.clip(1.0 - alpha, 1e-10, 1.0))

    p_r[0, 0] = alpha
    p_r[1, 0] = alpha * t0
    for cch in range(3):
        zc = (x * w1_r[0, cch] + y * w1_r[1, cch] + z * w1_r[2, cch]
              + dx * w2_r[0, cch] + dy * w2_r[1, cch] + dz * w2_r[2, cch])
        rgb = 1.0 / (1.0 + jnp.exp(-zc))
        p_r[2 + cch, 0] = alpha * (rgb - 1.0)


def _tc_pointwise(xyz_t, dir_t, ts_t, w1, w2):
    bs3 = pl.BlockSpec((3, 1, RR, 128), lambda i: (0, i, 0, 0))
    bs2 = pl.BlockSpec((2, 1, RR, 128), lambda i: (0, i, 0, 0))
    bs_sm = pl.BlockSpec(memory_space=pltpu.SMEM)
    return pl.pallas_call(
        _ptwise_kernel,
        grid=(NB,),
        in_specs=[bs3, bs3, bs2, bs_sm, bs_sm],
        out_specs=(
            pl.BlockSpec((1, RR, 128), lambda i: (i, 0, 0)),
            pl.BlockSpec((5, 1, RR, 128), lambda i: (0, i, 0, 0)),
        ),
        out_shape=(
            jax.ShapeDtypeStruct((NB, RR, 128), jnp.float32),
            jax.ShapeDtypeStruct((5, NB, RR, 128), jnp.float32),
        ),
    )(xyz_t, dir_t, ts_t, w1, w2)


def _scan1_kernel(x_r, o_r, acc):
    j = pl.program_id(0)

    @pl.when(j == 0)
    def _():
        acc[...] = jnp.zeros_like(acc)

    acc[...] = acc[...] + x_r[0]
    o_r[0] = acc[...]


def _tc_scan1(x2):
    x3 = x2.reshape(128, NCHUNK // 128, 128)
    return pl.pallas_call(
        _scan1_kernel,
        grid=(128,),
        in_specs=[pl.BlockSpec((1, NCHUNK // 128, 128), lambda j: (j, 0, 0))],
        out_specs=pl.BlockSpec((1, NCHUNK // 128, 128), lambda j: (j, 0, 0)),
        out_shape=jax.ShapeDtypeStruct((128, NCHUNK // 128, 128), jnp.float32),
        scratch_shapes=[pltpu.VMEM((NCHUNK // 128, 128), jnp.float32)],
        compiler_params=pltpu.CompilerParams(
            dimension_semantics=("arbitrary",),
        ),
    )(x3).reshape(128, NCHUNK)


def _scan23_kernel(rs_r, p1_r, acc):
    def step(j, _):
        acc[0, :] = acc[0, :] + rs_r[j, :]
        p1_r[j, :] = acc[0, :]
        return 0

    acc[0, :] = jnp.zeros((NGRP,), jnp.float32)
    lax.fori_loop(0, 128, step, 0)

    lane = lax.broadcasted_iota(jnp.int32, (1, NGRP), 1)
    vals = acc[0:1, :]

    def shr(a):
        return jnp.concatenate(
            [jnp.zeros((1, 1), jnp.float32), a[:, : NGRP - 1]], axis=1)

    for g in range(1, NGRP):
        vals = vals + jnp.where(lane == g, shr(vals), 0.0)
    e3 = shr(vals)
    p1_r[...] = p1_r[...] + e3


def _tc_scan23(rs2):
    return pl.pallas_call(
        _scan23_kernel,
        in_specs=[pl.BlockSpec((128, NGRP), lambda: (0, 0))],
        out_specs=pl.BlockSpec((128, NGRP), lambda: (0, 0)),
        out_shape=jax.ShapeDtypeStruct((128, NGRP), jnp.float32),
        scratch_shapes=[pltpu.VMEM((1, NGRP), jnp.float32)],
    )(rs2)


def _combine_kernel(i1_r, e_r, g_r, cnt_r, ex_r, seg_r, scr):
    i = pl.program_id(0)

    @pl.when(i == 0)
    def _():
        scr[0] = 0.0

    cs = i1_r[0] + e_r[0, 0][:, None]
    ex_r[0] = cs - g_r[0]

    c = cnt_r[0]
    cf = c.astype(jnp.float32)
    ii = lax.broadcasted_iota(jnp.int32, (128, 128), 0)
    jj = lax.broadcasted_iota(jnp.int32, (128, 128), 1)
    ut = (ii <= jj).astype(jnp.float32)
    ri = lax.broadcasted_iota(jnp.int32, (RR, RR), 0)
    rj = lax.broadcasted_iota(jnp.int32, (RR, RR), 1)
    lt = (rj < ri).astype(jnp.float32)
    cl = jnp.dot(cf, ut, preferred_element_type=jnp.float32)
    coffs = jnp.dot(lt, cl[:, 127:128], preferred_element_type=jnp.float32)
    cinc = cl + coffs
    seg_r[0] = (cinc + scr[0] - 1.0).astype(jnp.int32)
    scr[0] = scr[0] + jnp.sum(cf)


def _tc_combine(incl1, e3d, log1m, cnt):
    bs = pl.BlockSpec((1, RR, 128), lambda i: (i, 0, 0))
    return pl.pallas_call(
        _combine_kernel,
        grid=(NB,),
        in_specs=[bs, pl.BlockSpec((1, 1, RR), lambda i: (i, 0, 0)), bs, bs],
        out_specs=(bs, bs),
        out_shape=(
            jax.ShapeDtypeStruct((NB, RR, 128), jnp.float32),
            jax.ShapeDtypeStruct((NB, RR, 128), jnp.int32),
        ),
        scratch_shapes=[pltpu.SMEM((1,), jnp.float32)],
        compiler_params=pltpu.CompilerParams(
            dimension_semantics=("arbitrary",),
        ),
    )(incl1, e3d, log1m, cnt)


def _sc_reduce(excl, chans, seg, cu_pad):
    mesh = plsc.VectorSubcoreMesh(core_axis_name="c", subcore_axis_name="s")
    out_type = (
        jax.ShapeDtypeStruct((3 * N,), jnp.float32),
        jax.ShapeDtypeStruct((N,), jnp.float32),
        jax.ShapeDtypeStruct((N,), jnp.float32),
    )

    @functools.partial(
        pl.kernel,
        out_type=out_type,
        mesh=mesh,
        scratch_types=[
            pltpu.VMEM((16,), jnp.int32),
            pltpu.VMEM((16,), jnp.int32),
            pltpu.VMEM((RPT,), jnp.int32),
            pltpu.VMEM((RPT,), jnp.float32),
            pltpu.VMEM((CH,), jnp.int32),
            pltpu.VMEM((CH,), jnp.float32),
            [pltpu.VMEM((CH,), jnp.float32) for _ in range(5)],
            [pltpu.VMEM((RPT,), jnp.float32) for _ in range(5)],
            pltpu.SemaphoreType.DMA,
        ],
        compiler_params=pltpu.CompilerParams(needs_layout_passes=False),
    )
    def k(ex_hbm, ch_hbm, seg_hbm, cu_hbm, img_hbm, dep_hbm, ws_hbm,
          lo_v, hi_v, st_v, base_v, seg_v, ex_v, chbufs, accs, sem):
        cid = lax.axis_index("c")
        sid = lax.axis_index("s")
        w = cid * 16 + sid
        base_ray = w * RPT

        pltpu.sync_copy(cu_hbm.at[pl.ds(w * RPT, 16)], lo_v)
        pltpu.sync_copy(cu_hbm.at[pl.ds((w + 1) * RPT, 16)], hi_v)
        lo = lo_v[...][0]
        hi = hi_v[...][0]
        k0 = lo // CH
        k1 = (hi + CH - 1) // CH

        pltpu.sync_copy(cu_hbm.at[pl.ds(w * RPT, RPT)], st_v)
        pltpu.async_copy(ex_hbm.at[st_v], base_v, sem).wait()

        z16 = jnp.zeros((16,), jnp.float32)

        def zero(i, _):
            for a in accs:
                a[pl.ds(i * 16, 16)] = z16
            return 0

        lax.fori_loop(0, RPT // 16, zero, 0)

        def chunk(kc, _):
            cps = [pltpu.async_copy(seg_hbm.at[pl.ds(kc * CH, CH)], seg_v,
                                    sem),
                   pltpu.async_copy(ex_hbm.at[pl.ds(kc * CH, CH)], ex_v,
                                    sem)]
            for ci in range(5):
                cps.append(pltpu.async_copy(
                    ch_hbm.at[pl.ds(ci * T + kc * CH, CH)], chbufs[ci], sem))
            for cp in cps:
                cp.wait()

            def inner(j, _):
                s = seg_v[pl.ds(j * 16, 16)]
                rel = s - base_ray
                msk = (rel >= 0) & (rel < RPT)
                relc = jnp.where(msk, rel, 0)
                b = plsc.load_gather(base_v, [relc])
                trans = jnp.exp(ex_v[pl.ds(j * 16, 16)] - b)
                for ci in range(5):
                    v = chbufs[ci][pl.ds(j * 16, 16)] * trans
                    plsc.addupdate_scatter(accs[ci], [relc], v, mask=msk)
                return 0

            lax.fori_loop(0, CH // 16, inner, 0)
            return 0

        lax.fori_loop(k0, k1, chunk, 0)

        one16 = jnp.ones((16,), jnp.float32)

        def addone(i, _):
            for a in accs[2:]:
                a[pl.ds(i * 16, 16)] = a[pl.ds(i * 16, 16)] + one16
            return 0

        lax.fori_loop(0, RPT // 16, addone, 0)

        pltpu.sync_copy(accs[0], ws_hbm.at[pl.ds(base_ray, RPT)])
        pltpu.sync_copy(accs[1], dep_hbm.at[pl.ds(base_ray, RPT)])
        for ci in range(3):
            pltpu.sync_copy(accs[2 + ci],
                            img_hbm.at[pl.ds(ci * N + base_ray, RPT)])

    return k(excl, chans, seg, cu_pad)


def kernel(xyzs, dirs, ts, cu_seqlens, W1, W2):
    xyz_t = xyzs.T.reshape(3, NB, RR, 128)
    dir_t = dirs.T.reshape(3, NB, RR, 128)
    ts_t = ts.T.reshape(2, NB, RR, 128)
    cu_pad = jnp.concatenate(
        [cu_seqlens, jnp.full((15,), T, jnp.int32)])

    cnt = _sc_counts(cu_pad).reshape(NB, RR, 128)
    log1m, chans = _tc_pointwise(xyz_t, dir_t, ts_t, W1, W2)

    x2 = log1m.reshape(NCHUNK, 128).T
    cs1t = _tc_scan1(x2)
    incl1 = cs1t.T.reshape(NB, RR, 128)
    rs2 = cs1t[127].reshape(NGRP, 128).T
    p1t = _tc_scan23(rs2)
    p1 = p1t.T.reshape(NCHUNK)
    e = jnp.concatenate([jnp.zeros((1,), jnp.float32), p1[:-1]])
    e3d = e.reshape(NB, 1, RR)

    excl, seg = _tc_combine(incl1, e3d, log1m, cnt)

    img_t, depth, ws = _sc_reduce(
        excl.reshape(T), chans.reshape(5 * T), seg.reshape(T), cu_pad)
    return img_t.reshape(3, N).T, depth, ws


# speedup vs baseline: 201.5683x; 1.0587x over previous
import functools

import jax
import jax.numpy as jnp
from jax import lax
from jax.experimental import pallas as pl
from jax.experimental.pallas import tpu as pltpu
from jax.experimental.pallas import tpu_sc as plsc

T = 1048576
N = 65536

BLK = 2048
RR = BLK // 128
NB = T // BLK
NCHUNK = T // 128
NGRP = NCHUNK // 128

NTILE = 32
RPT = N // NTILE
CH = 4096
BIG = 3.0e38

HT = T // 2
SLC = HT // 16
EPS0 = N // 16


def _sc_counts(cu_pad):
    mesh = plsc.VectorSubcoreMesh(core_axis_name="c", subcore_axis_name="s")

    @functools.partial(
        pl.kernel,
        out_type=jax.ShapeDtypeStruct((T,), jnp.int32),
        mesh=mesh,
        scratch_types=[
            pltpu.VMEM((EPS0,), jnp.int32),
            pltpu.VMEM((EPS0,), jnp.int32),
            pltpu.VMEM((EPS0,), jnp.int32),
            pltpu.VMEM((SLC,), jnp.int32),
            pltpu.VMEM_SHARED((HT + 256,), jnp.int32),
        ],
        compiler_params=pltpu.CompilerParams(needs_layout_passes=False),
    )
    def k(cu_hbm, out_hbm, idx_v, idx2_v, ones_v, zbuf_v, cnt_sh):
        cid = lax.axis_index("c")
        sid = lax.axis_index("s")
        z16 = jnp.zeros((16,), jnp.int32)
        o16 = jnp.ones((16,), jnp.int32)

        def fill(i, _):
            zbuf_v[pl.ds(i * 16, 16)] = z16
            return 0

        lax.fori_loop(0, SLC // 16, fill, 0)

        def fill2(i, _):
            ones_v[pl.ds(i * 16, 16)] = o16
            return 0

        lax.fori_loop(0, EPS0 // 16, fill2, 0)

        pltpu.sync_copy(zbuf_v, cnt_sh.at[pl.ds(sid * SLC, SLC)])

        @pl.when(sid == 0)
        def _():
            pltpu.sync_copy(zbuf_v.at[pl.ds(0, 256)],
                            cnt_sh.at[pl.ds(HT, 256)])

        plsc.subcore_barrier()

        pltpu.sync_copy(cu_hbm.at[pl.ds(sid * EPS0, EPS0)], idx_v)
        lo = cid * HT

        def redirect(j, _):
            v = idx_v[pl.ds(j * 16, 16)]
            rel = v - lo
            msk = (rel >= 0) & (rel < HT)
            idx2_v[pl.ds(j * 16, 16)] = jnp.where(msk, rel, HT)
            return 0

        lax.fori_loop(0, EPS0 // 16, redirect, 0)
        pltpu.sync_copy(ones_v, cnt_sh.at[idx2_v], add=True)
        plsc.subcore_barrier()

        pltpu.sync_copy(
            cnt_sh.at[pl.ds(sid * SLC, SLC)],
            out_hbm.at[pl.ds(cid * HT + sid * SLC, SLC)],
        )

    return k(cu_pad)


def _ptwise_kernel(xyz_r, dir_r, ts_r, w1_r, w2_r, g_r, p_r):
    x = xyz_r[0, 0]
    y = xyz_r[1, 0]
    z = xyz_r[2, 0]
    dx = dir_r[0, 0]
    dy = dir_r[1, 0]
    dz = dir_r[2, 0]
    t0 = ts_r[0, 0]
    dt = ts_r[1, 0]

    d2 = (x * x + z * z) + y * y
    sig = 10.0 * jnp.exp(d2 * (-2.0))
    e2 = jnp.exp(-sig * dt)
    alpha = 1.0 - e2
    g_r[0] = jnp.log(jnp.clip(1.0 - alpha, 1e-10, 1.0))

    p_r[0, 0] = alpha
    p_r[1, 0] = alpha * t0
    for cch in range(3):
        zc = (x * w1_r[0, cch] + y * w1_r[1, cch] + z * w1_r[2, cch]
              + dx * w2_r[0, cch] + dy * w2_r[1, cch] + dz * w2_r[2, cch])
        rgb = 1.0 / (1.0 + jnp.exp(-zc))
        p_r[2 + cch, 0] = alpha * (rgb - 1.0)


def _tc_pointwise(xyz_t, dir_t, ts_t, w1, w2):
    bs3 = pl.BlockSpec((3, 1, RR, 128), lambda i: (0, i, 0, 0))
    bs2 = pl.BlockSpec((2, 1, RR, 128), lambda i: (0, i, 0, 0))
    bs_sm = pl.BlockSpec(memory_space=pltpu.SMEM)
    return pl.pallas_call(
        _ptwise_kernel,
        grid=(NB,),
        in_specs=[bs3, bs3, bs2, bs_sm, bs_sm],
        out_specs=(
            pl.BlockSpec((1, RR, 128), lambda i: (i, 0, 0)),
            pl.BlockSpec((5, 1, RR, 128), lambda i: (0, i, 0, 0)),
        ),
        out_shape=(
            jax.ShapeDtypeStruct((NB, RR, 128), jnp.float32),
            jax.ShapeDtypeStruct((5, NB, RR, 128), jnp.float32),
        ),
    )(xyz_t, dir_t, ts_t, w1, w2)


def _scan123_kernel(x_r, o_r, acc):
    acc[...] = jnp.zeros((NGRP, 128), jnp.float32)

    def step(j, _):
        acc[...] = acc[...] + x_r[j]
        o_r[j] = acc[...]
        return 0

    lax.fori_loop(0, 128, step, 0)

    lane = lax.broadcasted_iota(jnp.int32, (NGRP, 128), 1)
    vals = acc[...]

    def shr_lane(a):
        return jnp.concatenate(
            [jnp.zeros((NGRP, 1), jnp.float32), a[:, :127]], axis=1)

    for u in range(1, 128):
        vals = jnp.where(lane == u, vals + shr_lane(vals), vals)

    sub = lax.broadcasted_iota(jnp.int32, (NGRP, 1), 0)
    col = vals[:, 127:128]

    def shr_sub(a):
        return jnp.concatenate(
            [jnp.zeros((1, 1), jnp.float32), a[: NGRP - 1, :]], axis=0)

    for g in range(1, NGRP):
        col = jnp.where(sub == g, col + shr_sub(col), col)
    p1 = vals + shr_sub(col)
    e = jnp.concatenate([shr_sub(p1[:, 127:128]), p1[:, :127]], axis=1)

    def combine(j, _):
        o_r[j] = o_r[j] + e
        return 0

    lax.fori_loop(0, 128, combine, 0)


def _tc_scan123(x2):
    x3 = x2.reshape(128, NGRP, 128)
    return pl.pallas_call(
        _scan123_kernel,
        in_specs=[pl.BlockSpec((128, NGRP, 128), lambda: (0, 0, 0))],
        out_specs=pl.BlockSpec((128, NGRP, 128), lambda: (0, 0, 0)),
        out_shape=jax.ShapeDtypeStruct((128, NGRP, 128), jnp.float32),
        scratch_shapes=[pltpu.VMEM((NGRP, 128), jnp.float32)],
    )(x3).reshape(128, NCHUNK)


def _combine_kernel(cs_r, g_r, cnt_r, ex_r, seg_r, scr):
    i = pl.program_id(0)

    @pl.when(i == 0)
    def _():
        scr[0] = 0.0

    ex_r[0] = cs_r[0] - g_r[0]

    c = cnt_r[0]
    cf = c.astype(jnp.float32)
    ii = lax.broadcasted_iota(jnp.int32, (128, 128), 0)
    jj = lax.broadcasted_iota(jnp.int32, (128, 128), 1)
    ut = (ii <= jj).astype(jnp.float32)
    ri = lax.broadcasted_iota(jnp.int32, (RR, RR), 0)
    rj = lax.broadcasted_iota(jnp.int32, (RR, RR), 1)
    lt = (rj < ri).astype(jnp.float32)
    cl = jnp.dot(cf, ut, preferred_element_type=jnp.float32)
    coffs = jnp.dot(lt, cl[:, 127:128], preferred_element_type=jnp.float32)
    cinc = cl + coffs
    seg_r[0] = (cinc + scr[0] - 1.0).astype(jnp.int32)
    scr[0] = scr[0] + jnp.sum(cf)


def _tc_combine(cs, log1m, cnt):
    bs = pl.BlockSpec((1, RR, 128), lambda i: (i, 0, 0))
    return pl.pallas_call(
        _combine_kernel,
        grid=(NB,),
        in_specs=[bs, bs, bs],
        out_specs=(bs, bs),
        out_shape=(
            jax.ShapeDtypeStruct((NB, RR, 128), jnp.float32),
            jax.ShapeDtypeStruct((NB, RR, 128), jnp.int32),
        ),
        scratch_shapes=[pltpu.SMEM((1,), jnp.float32)],
        compiler_params=pltpu.CompilerParams(
            dimension_semantics=("arbitrary",),
        ),
    )(cs, log1m, cnt)


def _sc_reduce(excl, chans, seg, cu_pad):
    mesh = plsc.VectorSubcoreMesh(core_axis_name="c", subcore_axis_name="s")
    out_type = (
        jax.ShapeDtypeStruct((3 * N,), jnp.float32),
        jax.ShapeDtypeStruct((N,), jnp.float32),
        jax.ShapeDtypeStruct((N,), jnp.float32),
    )

    @functools.partial(
        pl.kernel,
        out_type=out_type,
        mesh=mesh,
        scratch_types=[
            pltpu.VMEM((16,), jnp.int32),
            pltpu.VMEM((16,), jnp.int32),
            pltpu.VMEM((RPT,), jnp.int32),
            pltpu.VMEM((RPT,), jnp.float32),
            pltpu.VMEM((CH,), jnp.int32),
            pltpu.VMEM((CH,), jnp.float32),
            [pltpu.VMEM((CH,), jnp.float32) for _ in range(5)],
            [pltpu.VMEM((RPT,), jnp.float32) for _ in range(5)],
            pltpu.SemaphoreType.DMA,
        ],
        compiler_params=pltpu.CompilerParams(needs_layout_passes=False),
    )
    def k(ex_hbm, ch_hbm, seg_hbm, cu_hbm, img_hbm, dep_hbm, ws_hbm,
          lo_v, hi_v, st_v, base_v, seg_v, ex_v, chbufs, accs, sem):
        cid = lax.axis_index("c")
        sid = lax.axis_index("s")
        w = cid * 16 + sid
        base_ray = w * RPT

        pltpu.sync_copy(cu_hbm.at[pl.ds(w * RPT, 16)], lo_v)
        pltpu.sync_copy(cu_hbm.at[pl.ds((w + 1) * RPT, 16)], hi_v)
        lo = lo_v[...][0]
        hi = hi_v[...][0]
        k0 = lo // CH
        k1 = (hi + CH - 1) // CH

        pltpu.sync_copy(cu_hbm.at[pl.ds(w * RPT, RPT)], st_v)
        pltpu.async_copy(ex_hbm.at[st_v], base_v, sem).wait()

        z16 = jnp.zeros((16,), jnp.float32)

        def zero(i, _):
            for a in accs:
                a[pl.ds(i * 16, 16)] = z16
            return 0

        lax.fori_loop(0, RPT // 16, zero, 0)

        def chunk(kc, _):
            cps = [pltpu.async_copy(seg_hbm.at[pl.ds(kc * CH, CH)], seg_v,
                                    sem),
                   pltpu.async_copy(ex_hbm.at[pl.ds(kc * CH, CH)], ex_v,
                                    sem)]
            for ci in range(5):
                cps.append(pltpu.async_copy(
                    ch_hbm.at[pl.ds(ci * T + kc * CH, CH)], chbufs[ci], sem))
            for cp in cps:
                cp.wait()

            def inner(j, _):
                s = seg_v[pl.ds(j * 16, 16)]
                rel = s - base_ray
                msk = (rel >= 0) & (rel < RPT)
                relc = jnp.where(msk, rel, 0)
                b = plsc.load_gather(base_v, [relc])
                trans = jnp.exp(ex_v[pl.ds(j * 16, 16)] - b)
                for ci in range(5):
                    v = chbufs[ci][pl.ds(j * 16, 16)] * trans
                    plsc.addupdate_scatter(accs[ci], [relc], v, mask=msk)
                return 0

            lax.fori_loop(0, CH // 16, inner, 0)
            return 0

        lax.fori_loop(k0, k1, chunk, 0)

        one16 = jnp.ones((16,), jnp.float32)

        def addone(i, _):
            for a in accs[2:]:
                a[pl.ds(i * 16, 16)] = a[pl.ds(i * 16, 16)] + one16
            return 0

        lax.fori_loop(0, RPT // 16, addone, 0)

        pltpu.sync_copy(accs[0], ws_hbm.at[pl.ds(base_ray, RPT)])
        pltpu.sync_copy(accs[1], dep_hbm.at[pl.ds(base_ray, RPT)])
        for ci in range(3):
            pltpu.sync_copy(accs[2 + ci],
                            img_hbm.at[pl.ds(ci * N + base_ray, RPT)])

    return k(excl, chans, seg, cu_pad)


def kernel(xyzs, dirs, ts, cu_seqlens, W1, W2):
    xyz_t = xyzs.T.reshape(3, NB, RR, 128)
    dir_t = dirs.T.reshape(3, NB, RR, 128)
    ts_t = ts.T.reshape(2, NB, RR, 128)
    cu_pad = jnp.concatenate(
        [cu_seqlens, jnp.full((15,), T, jnp.int32)])

    cnt = _sc_counts(cu_pad).reshape(NB, RR, 128)
    log1m, chans = _tc_pointwise(xyz_t, dir_t, ts_t, W1, W2)

    x2 = log1m.reshape(NCHUNK, 128).T
    cs2t = _tc_scan123(x2)
    cs = cs2t.T.reshape(NB, RR, 128)

    excl, seg = _tc_combine(cs, log1m, cnt)

    img_t, depth, ws = _sc_reduce(
        excl.reshape(T), chans.reshape(5 * T), seg.reshape(T), cu_pad)
    return img_t.reshape(3, N).T, depth, ws


# speedup vs baseline: 418.1564x; 2.0745x over previous
import functools

import jax
import jax.numpy as jnp
from jax import lax
from jax.experimental import pallas as pl
from jax.experimental.pallas import tpu as pltpu
from jax.experimental.pallas import tpu_sc as plsc

T = 1048576
N = 65536

BLK = 16384
RR = BLK // 128
NB = T // BLK
NCHUNK = T // 128
NGRP = NCHUNK // 128

NTILE = 32
RPT = N // NTILE
CH = 4096
BIG = 3.0e38

HT = T // 2
SLC = HT // 16
EPS0 = N // 16


def _sc_counts(cu_pad):
    mesh = plsc.VectorSubcoreMesh(core_axis_name="c", subcore_axis_name="s")

    @functools.partial(
        pl.kernel,
        out_type=jax.ShapeDtypeStruct((T,), jnp.int32),
        mesh=mesh,
        scratch_types=[
            pltpu.VMEM((EPS0,), jnp.int32),
            pltpu.VMEM((EPS0,), jnp.int32),
            pltpu.VMEM((EPS0,), jnp.int32),
            pltpu.VMEM((SLC,), jnp.int32),
            pltpu.VMEM_SHARED((HT + 256,), jnp.int32),
        ],
        compiler_params=pltpu.CompilerParams(needs_layout_passes=False),
    )
    def k(cu_hbm, out_hbm, idx_v, idx2_v, ones_v, zbuf_v, cnt_sh):
        cid = lax.axis_index("c")
        sid = lax.axis_index("s")
        z16 = jnp.zeros((16,), jnp.int32)
        o16 = jnp.ones((16,), jnp.int32)

        def fill(i, _):
            zbuf_v[pl.ds(i * 16, 16)] = z16
            return 0

        lax.fori_loop(0, SLC // 16, fill, 0)

        def fill2(i, _):
            ones_v[pl.ds(i * 16, 16)] = o16
            return 0

        lax.fori_loop(0, EPS0 // 16, fill2, 0)

        pltpu.sync_copy(zbuf_v, cnt_sh.at[pl.ds(sid * SLC, SLC)])

        @pl.when(sid == 0)
        def _():
            pltpu.sync_copy(zbuf_v.at[pl.ds(0, 256)],
                            cnt_sh.at[pl.ds(HT, 256)])

        plsc.subcore_barrier()

        pltpu.sync_copy(cu_hbm.at[pl.ds(sid * EPS0, EPS0)], idx_v)
        lo = cid * HT

        def redirect(j, _):
            v = idx_v[pl.ds(j * 16, 16)]
            rel = v - lo
            msk = (rel >= 0) & (rel < HT)
            idx2_v[pl.ds(j * 16, 16)] = jnp.where(msk, rel, HT)
            return 0

        lax.fori_loop(0, EPS0 // 16, redirect, 0)
        pltpu.sync_copy(ones_v, cnt_sh.at[idx2_v], add=True)
        plsc.subcore_barrier()

        pltpu.sync_copy(
            cnt_sh.at[pl.ds(sid * SLC, SLC)],
            out_hbm.at[pl.ds(cid * HT + sid * SLC, SLC)],
        )

    return k(cu_pad)


def _ptwise_kernel(xyz_r, dir_r, ts_r, w1_r, w2_r, g_r, p_r):
    x = xyz_r[0, 0]
    y = xyz_r[1, 0]
    z = xyz_r[2, 0]
    dx = dir_r[0, 0]
    dy = dir_r[1, 0]
    dz = dir_r[2, 0]
    t0 = ts_r[0, 0]
    dt = ts_r[1, 0]

    d2 = (x * x + z * z) + y * y
    sig = 10.0 * jnp.exp(d2 * (-2.0))
    e2 = jnp.exp(-sig * dt)
    alpha = 1.0 - e2
    g = jnp.log(jnp.clip(1.0 - alpha, 1e-10, 1.0))
    g_r[...] = g.T

    p_r[0, 0] = alpha
    p_r[1, 0] = alpha * t0
    for cch in range(3):
        zc = (x * w1_r[0, cch] + y * w1_r[1, cch] + z * w1_r[2, cch]
              + dx * w2_r[0, cch] + dy * w2_r[1, cch] + dz * w2_r[2, cch])
        rgb = 1.0 / (1.0 + jnp.exp(-zc))
        p_r[2 + cch, 0] = alpha * (rgb - 1.0)


def _tc_pointwise(xyz_t, dir_t, ts_t, w1, w2):
    bs3 = pl.BlockSpec((3, 1, RR, 128), lambda i: (0, i, 0, 0))
    bs2 = pl.BlockSpec((2, 1, RR, 128), lambda i: (0, i, 0, 0))
    bs_sm = pl.BlockSpec(memory_space=pltpu.SMEM)
    return pl.pallas_call(
        _ptwise_kernel,
        grid=(NB,),
        in_specs=[bs3, bs3, bs2, bs_sm, bs_sm],
        out_specs=(
            pl.BlockSpec((128, RR), lambda i: (0, i)),
            pl.BlockSpec((5, 1, RR, 128), lambda i: (0, i, 0, 0)),
        ),
        out_shape=(
            jax.ShapeDtypeStruct((128, NCHUNK), jnp.float32),
            jax.ShapeDtypeStruct((5, NB, RR, 128), jnp.float32),
        ),
    )(xyz_t, dir_t, ts_t, w1, w2)


def _scan123_kernel(x_r, o_r, acc):
    acc[...] = jnp.zeros((NGRP, 128), jnp.float32)

    def step(j, _):
        acc[...] = acc[...] + x_r[j]
        o_r[j] = acc[...]
        return 0

    lax.fori_loop(0, 128, step, 0)

    lane = lax.broadcasted_iota(jnp.int32, (NGRP, 128), 1)
    vals = acc[...]

    def shr_lane(a):
        return jnp.concatenate(
            [jnp.zeros((NGRP, 1), jnp.float32), a[:, :127]], axis=1)

    for u in range(1, 128):
        vals = jnp.where(lane == u, vals + shr_lane(vals), vals)

    sub = lax.broadcasted_iota(jnp.int32, (NGRP, 1), 0)
    col = vals[:, 127:128]

    def shr_sub(a):
        return jnp.concatenate(
            [jnp.zeros((1, 1), jnp.float32), a[: NGRP - 1, :]], axis=0)

    for g in range(1, NGRP):
        col = jnp.where(sub == g, col + shr_sub(col), col)
    p1 = vals + shr_sub(col)
    e = jnp.concatenate([shr_sub(p1[:, 127:128]), p1[:, :127]], axis=1)

    def combine(j, _):
        o_r[j] = o_r[j] + e
        return 0

    lax.fori_loop(0, 128, combine, 0)


def _tc_scan123(x2):
    x3 = x2.reshape(128, NGRP, 128)
    return pl.pallas_call(
        _scan123_kernel,
        in_specs=[pl.BlockSpec((128, NGRP, 128), lambda: (0, 0, 0))],
        out_specs=pl.BlockSpec((128, NGRP, 128), lambda: (0, 0, 0)),
        out_shape=jax.ShapeDtypeStruct((128, NGRP, 128), jnp.float32),
        scratch_shapes=[pltpu.VMEM((NGRP, 128), jnp.float32)],
    )(x3).reshape(128, NCHUNK)


def _combine_kernel(cs_r, g_r, cnt_r, ex_r, seg_r, scr):
    i = pl.program_id(0)

    @pl.when(i == 0)
    def _():
        scr[0] = 0.0

    ex_r[0] = (cs_r[...] - g_r[...]).T

    c = cnt_r[0]
    cf = c.astype(jnp.float32)
    ii = lax.broadcasted_iota(jnp.int32, (128, 128), 0)
    jj = lax.broadcasted_iota(jnp.int32, (128, 128), 1)
    ut = (ii <= jj).astype(jnp.float32)
    ri = lax.broadcasted_iota(jnp.int32, (RR, RR), 0)
    rj = lax.broadcasted_iota(jnp.int32, (RR, RR), 1)
    lt = (rj < ri).astype(jnp.float32)
    cl = jnp.dot(cf, ut, preferred_element_type=jnp.float32)
    coffs = jnp.dot(lt, cl[:, 127:128], preferred_element_type=jnp.float32)
    cinc = cl + coffs
    seg_r[0] = (cinc + scr[0] - 1.0).astype(jnp.int32)
    scr[0] = scr[0] + jnp.sum(cf)


def _tc_combine(cs_t, log1m_t, cnt):
    bs = pl.BlockSpec((1, RR, 128), lambda i: (i, 0, 0))
    bst = pl.BlockSpec((128, RR), lambda i: (0, i))
    return pl.pallas_call(
        _combine_kernel,
        grid=(NB,),
        in_specs=[bst, bst, bs],
        out_specs=(bs, bs),
        out_shape=(
            jax.ShapeDtypeStruct((NB, RR, 128), jnp.float32),
            jax.ShapeDtypeStruct((NB, RR, 128), jnp.int32),
        ),
        scratch_shapes=[pltpu.SMEM((1,), jnp.float32)],
        compiler_params=pltpu.CompilerParams(
            dimension_semantics=("arbitrary",),
        ),
    )(cs_t, log1m_t, cnt)


def _sc_reduce(excl, chans, seg, cu_pad):
    mesh = plsc.VectorSubcoreMesh(core_axis_name="c", subcore_axis_name="s")
    out_type = (
        jax.ShapeDtypeStruct((3 * N,), jnp.float32),
        jax.ShapeDtypeStruct((N,), jnp.float32),
        jax.ShapeDtypeStruct((N,), jnp.float32),
    )

    @functools.partial(
        pl.kernel,
        out_type=out_type,
        mesh=mesh,
        scratch_types=[
            pltpu.VMEM((16,), jnp.int32),
            pltpu.VMEM((16,), jnp.int32),
            pltpu.VMEM((RPT,), jnp.int32),
            pltpu.VMEM((RPT,), jnp.float32),
            pltpu.VMEM((CH,), jnp.int32),
            pltpu.VMEM((CH,), jnp.float32),
            [pltpu.VMEM((CH,), jnp.float32) for _ in range(5)],
            [pltpu.VMEM((RPT,), jnp.float32) for _ in range(5)],
            pltpu.SemaphoreType.DMA,
        ],
        compiler_params=pltpu.CompilerParams(needs_layout_passes=False),
    )
    def k(ex_hbm, ch_hbm, seg_hbm, cu_hbm, img_hbm, dep_hbm, ws_hbm,
          lo_v, hi_v, st_v, base_v, seg_v, ex_v, chbufs, accs, sem):
        cid = lax.axis_index("c")
        sid = lax.axis_index("s")
        w = cid * 16 + sid
        base_ray = w * RPT

        pltpu.sync_copy(cu_hbm.at[pl.ds(w * RPT, 16)], lo_v)
        pltpu.sync_copy(cu_hbm.at[pl.ds((w + 1) * RPT, 16)], hi_v)
        lo = lo_v[...][0]
        hi = hi_v[...][0]
        k0 = lo // CH
        k1 = (hi + CH - 1) // CH

        pltpu.sync_copy(cu_hbm.at[pl.ds(w * RPT, RPT)], st_v)
        pltpu.async_copy(ex_hbm.at[st_v], base_v, sem).wait()

        z16 = jnp.zeros((16,), jnp.float32)

        def zero(i, _):
            for a in accs:
                a[pl.ds(i * 16, 16)] = z16
            return 0

        lax.fori_loop(0, RPT // 16, zero, 0)

        def chunk(kc, _):
            cps = [pltpu.async_copy(seg_hbm.at[pl.ds(kc * CH, CH)], seg_v,
                                    sem),
                   pltpu.async_copy(ex_hbm.at[pl.ds(kc * CH, CH)], ex_v,
                                    sem)]
            for ci in range(5):
                cps.append(pltpu.async_copy(
                    ch_hbm.at[pl.ds(ci * T + kc * CH, CH)], chbufs[ci], sem))
            for cp in cps:
                cp.wait()

            def inner(j, _):
                s = seg_v[pl.ds(j * 16, 16)]
                rel = s - base_ray
                msk = (rel >= 0) & (rel < RPT)
                relc = jnp.where(msk, rel, 0)
                b = plsc.load_gather(base_v, [relc])
                trans = jnp.exp(ex_v[pl.ds(j * 16, 16)] - b)
                for ci in range(5):
                    v = chbufs[ci][pl.ds(j * 16, 16)] * trans
                    plsc.addupdate_scatter(accs[ci], [relc], v, mask=msk)
                return 0

            lax.fori_loop(0, CH // 16, inner, 0)
            return 0

        lax.fori_loop(k0, k1, chunk, 0)

        one16 = jnp.ones((16,), jnp.float32)

        def addone(i, _):
            for a in accs[2:]:
                a[pl.ds(i * 16, 16)] = a[pl.ds(i * 16, 16)] + one16
            return 0

        lax.fori_loop(0, RPT // 16, addone, 0)

        pltpu.sync_copy(accs[0], ws_hbm.at[pl.ds(base_ray, RPT)])
        pltpu.sync_copy(accs[1], dep_hbm.at[pl.ds(base_ray, RPT)])
        for ci in range(3):
            pltpu.sync_copy(accs[2 + ci],
                            img_hbm.at[pl.ds(ci * N + base_ray, RPT)])

    return k(excl, chans, seg, cu_pad)


def kernel(xyzs, dirs, ts, cu_seqlens, W1, W2):
    xyz_t = xyzs.T.reshape(3, NB, RR, 128)
    dir_t = dirs.T.reshape(3, NB, RR, 128)
    ts_t = ts.T.reshape(2, NB, RR, 128)
    cu_pad = jnp.concatenate(
        [cu_seqlens, jnp.full((15,), T, jnp.int32)])

    cnt = _sc_counts(cu_pad).reshape(NB, RR, 128)
    log1m_t, chans = _tc_pointwise(xyz_t, dir_t, ts_t, W1, W2)

    cs2t = _tc_scan123(log1m_t)
    excl, seg = _tc_combine(cs2t, log1m_t, cnt)

    img_t, depth, ws = _sc_reduce(
        excl.reshape(T), chans.reshape(5 * T), seg.reshape(T), cu_pad)
    return img_t.reshape(3, N).T, depth, ws
